# speedup vs baseline: 6.5207x; 6.5207x over previous
import jax
import jax.numpy as jnp
from jax import lax
from jax.experimental import pallas as pl
from jax.experimental.pallas import tpu as pltpu
from jax.experimental.pallas import tpu_sc as plsc

N_NODES = 100000
IN_SIZE = 128
OUT_SIZE = 128
S = 10
B = 4096
M = B * (S + 1)

NW = 32
A_PER_W = B // NW
N_PER_W = (M - B) // NW
CHUNK = 64
A_CHUNKS = A_PER_W // CHUNK
N_CHUNKS = N_PER_W // CHUNK
IDX_PER_W = A_PER_W + N_PER_W
NIDX_PER_W = IDX_PER_W * S


def _sc_body(emb_hbm, nodes_hbm, neigh_hbm,
             self_s_hbm, agg_s_hbm, self_n_hbm, agg_n_hbm,
             idx_v, nidx_v, selfbuf, nbuf, aggbuf, sem):
    wid = lax.axis_index("s") * 2 + lax.axis_index("c")

    a_base = wid * A_PER_W
    n_base = wid * N_PER_W
    pltpu.sync_copy(nodes_hbm.at[pl.ds(a_base, A_PER_W)],
                    idx_v.at[pl.ds(0, A_PER_W)])
    pltpu.sync_copy(nodes_hbm.at[pl.ds(B + n_base, N_PER_W)],
                    idx_v.at[pl.ds(A_PER_W, N_PER_W)])
    pltpu.sync_copy(neigh_hbm.at[pl.ds(a_base * S, A_PER_W * S)],
                    nidx_v.at[pl.ds(0, A_PER_W * S)])
    pltpu.sync_copy(neigh_hbm.at[pl.ds((B + n_base) * S, N_PER_W * S)],
                    nidx_v.at[pl.ds(A_PER_W * S, N_PER_W * S)])

    def do_chunk(idx_off, out_self, out_agg, out_row):
        cps = [pltpu.async_copy(emb_hbm.at[idx_v.at[pl.ds(idx_off, CHUNK)]],
                                selfbuf, sem)]
        for q in range(CHUNK * S // 128):
            cps.append(pltpu.async_copy(
                emb_hbm.at[nidx_v.at[pl.ds(idx_off * S + q * 128, 128)]],
                nbuf.at[pl.ds(q * 128, 128), :], sem))
        for cp in cps:
            cp.wait()

        def row_body(i, carry):
            r = i * S
            for k in range(IN_SIZE // 16):
                acc = nbuf[r, pl.ds(k * 16, 16)]
                for j in range(1, S):
                    acc = acc + nbuf[r + j, pl.ds(k * 16, 16)]
                aggbuf[i, pl.ds(k * 16, 16)] = acc
            return carry
        lax.fori_loop(0, CHUNK, row_body, 0)

        pltpu.sync_copy(selfbuf, out_self.at[pl.ds(out_row, CHUNK), :])
        pltpu.sync_copy(aggbuf, out_agg.at[pl.ds(out_row, CHUNK), :])

    def a_chunk(c, carry):
        do_chunk(c * CHUNK, self_s_hbm, agg_s_hbm, a_base + c * CHUNK)
        return carry
    lax.fori_loop(0, A_CHUNKS, a_chunk, 0)

    def n_chunk(c, carry):
        do_chunk(A_PER_W + c * CHUNK, self_n_hbm, agg_n_hbm,
                 n_base + c * CHUNK)
        return carry
    lax.fori_loop(0, N_CHUNKS, n_chunk, 0)


def _sc_gather(emb_table, nodes1, neigh_flat):
    mesh = plsc.VectorSubcoreMesh(core_axis_name="c", subcore_axis_name="s")
    f = pl.kernel(
        _sc_body,
        out_type=[
            jax.ShapeDtypeStruct((B, IN_SIZE), jnp.float32),
            jax.ShapeDtypeStruct((B, IN_SIZE), jnp.float32),
            jax.ShapeDtypeStruct((M - B, IN_SIZE), jnp.float32),
            jax.ShapeDtypeStruct((M - B, IN_SIZE), jnp.float32),
        ],
        mesh=mesh,
        scratch_types=[
            pltpu.VMEM((IDX_PER_W,), jnp.int32),
            pltpu.VMEM((NIDX_PER_W,), jnp.int32),
            pltpu.VMEM((CHUNK, IN_SIZE), jnp.float32),
            pltpu.VMEM((CHUNK * S, IN_SIZE), jnp.float32),
            pltpu.VMEM((CHUNK, IN_SIZE), jnp.float32),
            pltpu.SemaphoreType.DMA,
        ],
    )
    return f(emb_table, nodes1, neigh_flat)


def _tc_body(ss, sa, ns, na, w1a, w1b, w2a, w2b, o):
    f32 = jnp.float32
    h1s = jnp.maximum(
        jnp.dot(ss[:], w1a[:], preferred_element_type=f32)
        + jnp.dot(sa[:], w1b[:], preferred_element_type=f32), 0.0)
    h1n = jnp.maximum(
        jnp.dot(ns[:], w1a[:], preferred_element_type=f32)
        + jnp.dot(na[:], w1b[:], preferred_element_type=f32), 0.0)
    agg1 = jnp.sum(h1n.reshape(h1s.shape[0], S, OUT_SIZE), axis=1)
    o[:] = jnp.maximum(
        jnp.dot(h1s, w2a[:], preferred_element_type=f32)
        + jnp.dot(agg1, w2b[:], preferred_element_type=f32), 0.0)


_TC_GRID = 16
_TB = B // _TC_GRID


def _tc_call(self_s, agg_s, self_n, agg_n, w1a, w1b, w2a, w2b):
    wspec = pl.BlockSpec((IN_SIZE, OUT_SIZE), lambda p: (0, 0))
    return pl.pallas_call(
        _tc_body,
        grid=(_TC_GRID,),
        in_specs=[
            pl.BlockSpec((_TB, IN_SIZE), lambda p: (p, 0)),
            pl.BlockSpec((_TB, IN_SIZE), lambda p: (p, 0)),
            pl.BlockSpec((_TB * S, IN_SIZE), lambda p: (p, 0)),
            pl.BlockSpec((_TB * S, IN_SIZE), lambda p: (p, 0)),
            wspec, wspec, wspec, wspec,
        ],
        out_specs=pl.BlockSpec((_TB, OUT_SIZE), lambda p: (p, 0)),
        out_shape=jax.ShapeDtypeStruct((B, OUT_SIZE), jnp.float32),
    )(self_s, agg_s, self_n, agg_n, w1a, w1b, w2a, w2b)


def kernel(emb_table, W1, W2, node_batch, nodes1, neigh1, neigh2):
    neigh_flat = neigh1.reshape(-1)
    self_s, agg_s, self_n, agg_n = _sc_gather(emb_table, nodes1, neigh_flat)
    w1a = W1[:, :IN_SIZE].T
    w1b = W1[:, IN_SIZE:].T * (1.0 / S)
    w2a = W2[:, :OUT_SIZE].T
    w2b = W2[:, OUT_SIZE:].T * (1.0 / S)
    return _tc_call(self_s, agg_s, self_n, agg_n, w1a, w1b, w2a, w2b)


# speedup vs baseline: 14.8306x; 2.2744x over previous
import jax
import jax.numpy as jnp
from jax import lax
from jax.experimental import pallas as pl
from jax.experimental.pallas import tpu as pltpu
from jax.experimental.pallas import tpu_sc as plsc

N_NODES = 100000
IN_SIZE = 128
OUT_SIZE = 128
S = 10
B = 4096
M = B * (S + 1)

NW = 32
A_PER_W = B // NW
N_PER_W = (M - B) // NW
CHUNK = 128
IDX_PER_W = A_PER_W + N_PER_W
NCHUNKS = IDX_PER_W // CHUNK


def _sc_body(emb_hbm, nodes_hbm, neight_hbm,
             self_s_hbm, agg_s_hbm, self_n_hbm, agg_n_hbm,
             idx_v, nidx_v, selfbuf, aggbuf, sem_g0, sem_g1, sem_w):
    wid = lax.axis_index("s") * 2 + lax.axis_index("c")
    sems_g = (sem_g0, sem_g1)

    a_base = wid * A_PER_W
    n_base = wid * N_PER_W
    pltpu.sync_copy(nodes_hbm.at[pl.ds(a_base, A_PER_W)],
                    idx_v.at[pl.ds(0, A_PER_W)])
    pltpu.sync_copy(nodes_hbm.at[pl.ds(B + n_base, N_PER_W)],
                    idx_v.at[pl.ds(A_PER_W, N_PER_W)])
    pltpu.sync_copy(neight_hbm.at[:, pl.ds(a_base, A_PER_W)],
                    nidx_v.at[:, pl.ds(0, A_PER_W)])
    pltpu.sync_copy(neight_hbm.at[:, pl.ds(B + n_base, N_PER_W)],
                    nidx_v.at[:, pl.ds(A_PER_W, N_PER_W)])

    zero16 = jnp.zeros((16,), jnp.float32)

    def zero_agg(buf):
        def zrow(i, carry):
            for k in range(IN_SIZE // 16):
                aggbuf[buf, i, pl.ds(k * 16, 16)] = zero16
            return carry
        lax.fori_loop(0, CHUNK, zrow, 0)

    def fire(c, buf):
        cps = [pltpu.async_copy(
            emb_hbm.at[idx_v.at[pl.ds(c * CHUNK, CHUNK)]],
            selfbuf.at[buf], sems_g[buf])]
        for j in range(S):
            cps.append(pltpu.async_copy(
                emb_hbm.at[nidx_v.at[j, pl.ds(c * CHUNK, CHUNK)]],
                aggbuf.at[buf], sems_g[buf], add=True))
        return cps

    def out_refs(c):
        if c == 0:
            return self_s_hbm, agg_s_hbm, a_base
        return self_n_hbm, agg_n_hbm, n_base + (c - 1) * CHUNK

    zero_agg(0)
    gath = {0: fire(0, 0)}
    writes = {}
    for c in range(NCHUNKS):
        buf = c % 2
        nxt = 1 - buf
        if c >= 1:
            for cp in writes[c - 1]:
                cp.wait()
        if c + 1 < NCHUNKS:
            zero_agg(nxt)
            gath[c + 1] = fire(c + 1, nxt)
        for cp in gath[c]:
            cp.wait()
        o_self, o_agg, row = out_refs(c)
        writes[c] = [
            pltpu.async_copy(selfbuf.at[buf],
                             o_self.at[pl.ds(row, CHUNK), :], sem_w),
            pltpu.async_copy(aggbuf.at[buf],
                             o_agg.at[pl.ds(row, CHUNK), :], sem_w),
        ]
    for cp in writes[NCHUNKS - 1]:
        cp.wait()


def _sc_gather(emb_table, nodes1, neigh_t):
    mesh = plsc.VectorSubcoreMesh(core_axis_name="c", subcore_axis_name="s")
    f = pl.kernel(
        _sc_body,
        out_type=[
            jax.ShapeDtypeStruct((B, IN_SIZE), jnp.float32),
            jax.ShapeDtypeStruct((B, IN_SIZE), jnp.float32),
            jax.ShapeDtypeStruct((M - B, IN_SIZE), jnp.float32),
            jax.ShapeDtypeStruct((M - B, IN_SIZE), jnp.float32),
        ],
        mesh=mesh,
        scratch_types=[
            pltpu.VMEM((IDX_PER_W,), jnp.int32),
            pltpu.VMEM((S, IDX_PER_W), jnp.int32),
            pltpu.VMEM((2, CHUNK, IN_SIZE), jnp.float32),
            pltpu.VMEM((2, CHUNK, IN_SIZE), jnp.float32),
            pltpu.SemaphoreType.DMA,
            pltpu.SemaphoreType.DMA,
            pltpu.SemaphoreType.DMA,
        ],
    )
    return f(emb_table, nodes1, neigh_t)


def _tc_body(ss, sa, ns, na, w1a, w1b, w2a, w2b, o):
    f32 = jnp.float32
    h1s = jnp.maximum(
        jnp.dot(ss[:], w1a[:], preferred_element_type=f32)
        + jnp.dot(sa[:], w1b[:], preferred_element_type=f32), 0.0)
    h1n = jnp.maximum(
        jnp.dot(ns[:], w1a[:], preferred_element_type=f32)
        + jnp.dot(na[:], w1b[:], preferred_element_type=f32), 0.0)
    agg1 = jnp.sum(h1n.reshape(h1s.shape[0], S, OUT_SIZE), axis=1)
    o[:] = jnp.maximum(
        jnp.dot(h1s, w2a[:], preferred_element_type=f32)
        + jnp.dot(agg1, w2b[:], preferred_element_type=f32), 0.0)


_TC_GRID = 16
_TB = B // _TC_GRID


def _tc_call(self_s, agg_s, self_n, agg_n, w1a, w1b, w2a, w2b):
    wspec = pl.BlockSpec((IN_SIZE, OUT_SIZE), lambda p: (0, 0))
    return pl.pallas_call(
        _tc_body,
        grid=(_TC_GRID,),
        in_specs=[
            pl.BlockSpec((_TB, IN_SIZE), lambda p: (p, 0)),
            pl.BlockSpec((_TB, IN_SIZE), lambda p: (p, 0)),
            pl.BlockSpec((_TB * S, IN_SIZE), lambda p: (p, 0)),
            pl.BlockSpec((_TB * S, IN_SIZE), lambda p: (p, 0)),
            wspec, wspec, wspec, wspec,
        ],
        out_specs=pl.BlockSpec((_TB, OUT_SIZE), lambda p: (p, 0)),
        out_shape=jax.ShapeDtypeStruct((B, OUT_SIZE), jnp.float32),
    )(self_s, agg_s, self_n, agg_n, w1a, w1b, w2a, w2b)


def kernel(emb_table, W1, W2, node_batch, nodes1, neigh1, neigh2):
    neigh_t = neigh1.T
    self_s, agg_s, self_n, agg_n = _sc_gather(emb_table, nodes1, neigh_t)
    w1a = W1[:, :IN_SIZE].T
    w1b = W1[:, IN_SIZE:].T * (1.0 / S)
    w2a = W2[:, :OUT_SIZE].T
    w2b = W2[:, OUT_SIZE:].T * (1.0 / S)
    return _tc_call(self_s, agg_s, self_n, agg_n, w1a, w1b, w2a, w2b)
